# baseline (speedup 1.0000x reference)
import functools

import jax
import jax.numpy as jnp
from jax import lax
from jax.experimental import pallas as pl
from jax.experimental.pallas import tpu as pltpu
from jax.experimental.pallas import tpu_sc as plsc

CHUNK = 128
C0 = 57344
SC_SLABS = 20
SC_W = 2048
C1 = C0 + SC_SLABS * SC_W
LOG2E = 1.4426950408889634
LN2 = 0.6931471805599453
P5 = (9.94950803270367e-06, 0.9992359203110714, -0.49023275322858,
      0.2852763732841391, -0.1315845699212208, 0.030449658164617362)


def _softplus_fast(x):
  e = jnp.exp2(jnp.abs(x) * (-LOG2E))
  return jnp.maximum(x, 0.0) + jnp.log2(1.0 + e) * LN2


def _pack_pair(u_lo, u_hi):
  return lax.bitcast_convert_type(
      (u_hi & jnp.uint32(0xFFFF0000)) | (u_lo >> 16), jnp.int32)


def _stream_pass_tc(logits, c_t=2048):
  b, c = logits.shape
  nhead = C0 // c_t
  ncb = nhead + 1
  pairs = c_t // (2 * CHUNK)
  cp = ncb * pairs
  valid_last = c - C1
  tail_blk = C1 // c_t

  def body(x_ref, acc_ref, tab_ref):
    j = pl.program_id(0)

    @pl.when(j == 0)
    def _():
      acc_ref[...] = jnp.zeros_like(acc_ref)

    x = x_ref[...]
    u = lax.bitcast_convert_type(x, jnp.uint32)
    for p in range(pairs):
      tab_ref[:, p, :] = _pack_pair(
          u[:, 2 * p * CHUNK:(2 * p + 1) * CHUNK],
          u[:, (2 * p + 1) * CHUNK:(2 * p + 2) * CHUNK])
    y = _softplus_fast(x)

    def accum(y):
      acc = acc_ref[...]
      for s in range(c_t // 128):
        acc = acc + y[:, s * 128:(s + 1) * 128]
      acc_ref[...] = acc

    @pl.when(j < ncb - 1)
    def _():
      accum(y)

    @pl.when(j == ncb - 1)
    def _():
      col = lax.broadcasted_iota(jnp.int32, (b, c_t), 1)
      accum(jnp.where(col < valid_last, y, 0.0))

  return pl.pallas_call(
      body,
      grid=(ncb,),
      in_specs=[pl.BlockSpec(
          (b, c_t), lambda j: (0, jnp.where(j < nhead, j, tail_blk)))],
      out_specs=[
          pl.BlockSpec((b, 128), lambda j: (0, 0)),
          pl.BlockSpec((b, pairs, CHUNK), lambda j: (0, j, 0)),
      ],
      out_shape=[
          jax.ShapeDtypeStruct((b, 128), jnp.float32),
          jax.ShapeDtypeStruct((b, cp, CHUNK), jnp.int32),
      ],
  )(logits)


def _stream_pass_sc(logits):
  b, c = logits.shape
  n_full = SC_SLABS
  cp_sc = n_full * (SC_W // (2 * CHUNK))
  info = plsc.get_sparse_core_info()
  nw = info.num_cores * info.num_subcores
  rows_w = b // nw
  mesh = plsc.VectorSubcoreMesh(core_axis_name="c", subcore_axis_name="s")

  def horner(e):
    r = jnp.full((16,), P5[-1], jnp.float32)
    for a in P5[-2::-1]:
      r = r * e + a
    return r

  @functools.partial(
      pl.kernel,
      mesh=mesh,
      out_type=[
          jax.ShapeDtypeStruct((b, 16), jnp.float32),
          jax.ShapeDtypeStruct((b, cp_sc, CHUNK), jnp.int32),
      ],
      scratch_types=[
          pltpu.VMEM((8, SC_W), jnp.float32),
          pltpu.VMEM((8, 8, CHUNK), jnp.int32),
          pltpu.VMEM((rows_w, 16), jnp.float32),
          pltpu.SemaphoreType.DMA,
      ],
  )
  def sc_kernel(lg_hbm, acc_hbm, tab_hbm, buf_v, pk_v, acc_v, sem):
    wid = lax.axis_index("s") * info.num_cores + lax.axis_index("c")
    row_base = pl.multiple_of(wid * rows_w, 8)
    for q in range(rows_w):
      acc_v[q, :] = jnp.zeros((16,), jnp.float32)

    def do_slab(g8, cc, fill, n_vec, n_pk):
      r0 = pl.multiple_of(row_base + g8 * 8, 8)
      col = C0 + cc * SC_W
      pltpu.sync_copy(
          lg_hbm.at[pl.ds(r0, 8), pl.ds(col, fill)],
          buf_v.at[:, pl.ds(0, fill)])

      def vstep(w, carry):
        for s in range(8):
          v = buf_v[s, pl.ds(w * 16, 16)]
          t = jnp.abs(v)
          e = jnp.exp(-t)
          sp = jnp.maximum(v, 0.0) + horner(e)
          acc_v[g8 * 8 + s, :] = acc_v[g8 * 8 + s, :] + sp
        return carry

      lax.fori_loop(0, n_vec, vstep, 0)

      def pstep(i, carry):
        p = i // 8
        t = i % 8
        for s in range(8):
          lo = lax.bitcast_convert_type(
              buf_v[s, pl.ds(2 * p * CHUNK + t * 16, 16)], jnp.uint32)
          hi = lax.bitcast_convert_type(
              buf_v[s, pl.ds((2 * p + 1) * CHUNK + t * 16, 16)], jnp.uint32)
          pk_v[s, p, pl.ds(t * 16, 16)] = _pack_pair(lo, hi)
        return carry

      lax.fori_loop(0, n_pk * 8, pstep, 0)
      pltpu.sync_copy(
          pk_v,
          tab_hbm.at[pl.ds(r0, 8), pl.ds(cc * (SC_W // (2 * CHUNK)), 8), :])

    for g8 in range(rows_w // 8):
      def full_slab(cc, carry):
        do_slab(g8, cc, SC_W, SC_W // 16, 8)
        return carry

      lax.fori_loop(0, n_full, full_slab, 0)

    pltpu.sync_copy(acc_v, acc_hbm.at[pl.ds(row_base, rows_w)])

  return sc_kernel(logits)


def _gather_chunks(table, chunk_idx):
  n = chunk_idx.shape[0]
  info = plsc.get_sparse_core_info()
  nw = info.num_cores * info.num_subcores
  per_w = n // nw
  assert n % (8 * nw) == 0
  mesh = plsc.VectorSubcoreMesh(core_axis_name="c", subcore_axis_name="s")

  @functools.partial(
      pl.kernel,
      mesh=mesh,
      out_type=jax.ShapeDtypeStruct((n, CHUNK), jnp.int32),
      scratch_types=[
          pltpu.VMEM((per_w,), jnp.int32),
          pltpu.VMEM((per_w, CHUNK), jnp.int32),
          pltpu.SemaphoreType.DMA,
      ],
  )
  def sc_kernel(tab_hbm, idx_hbm, out_hbm, idx_v, rows_v, sem):
    wid = lax.axis_index("s") * info.num_cores + lax.axis_index("c")
    base = wid * per_w
    pltpu.sync_copy(idx_hbm.at[pl.ds(base, per_w)], idx_v)
    pltpu.async_copy(tab_hbm.at[idx_v], rows_v, sem).wait()
    pltpu.sync_copy(rows_v, out_hbm.at[pl.ds(base, per_w)])

  return sc_kernel(table, chunk_idx)


def _combine(acc_tc, acc_sc, g1, g2, cand, c):
  b, k = cand.shape

  def body(atc_ref, asc_ref, g1_ref, g2_ref, cand_ref, out_ref):
    row_sum = (jnp.sum(atc_ref[...], axis=1, keepdims=True) +
               jnp.sum(asc_ref[...], axis=1, keepdims=True))
    cd = cand_ref[...]
    lanes = lax.broadcasted_iota(jnp.int32, (b, CHUNK), 1)
    hi_mask = jnp.int32(-65536)
    gs = []
    for kk in range(k):
      cdk = cd[:, kk:kk + 1]
      sel = lanes == (cdk % CHUNK)
      in_tc = (cdk < C0) | (cdk >= C1)
      row = jnp.where(
          sel,
          jnp.where(in_tc, g1_ref[:, kk * CHUNK:(kk + 1) * CHUNK],
                    g2_ref[:, kk * CHUNK:(kk + 1) * CHUNK]), 0)
      packed = jnp.sum(row, axis=1, keepdims=True)
      half_hi = (cdk // CHUNK) % 2 == 1
      bits = jnp.where(half_hi, packed & hi_mask, packed << 16)
      gs.append(lax.bitcast_convert_type(bits, jnp.float32))
    pos = gs[0]
    for kk in range(1, k):
      pos = pos + gs[kk]
    pos = pos / k
    sub = _softplus_fast(gs[0])
    n_uniq = jnp.ones((b, 1), jnp.float32)
    for kk in range(1, k):
      w = jnp.ones((b, 1), jnp.float32)
      for jj in range(kk):
        w = w * (cd[:, kk:kk + 1] != cd[:, jj:jj + 1]).astype(jnp.float32)
      sub = sub + w * _softplus_fast(gs[kk])
      n_uniq = n_uniq + w
    neg = (row_sum - sub) / (c - n_uniq)
    per = _softplus_fast(-pos) + neg
    out_ref[0, 0] = jnp.sum(per) / b

  return pl.pallas_call(
      body,
      out_specs=pl.BlockSpec(memory_space=pltpu.SMEM),
      out_shape=jax.ShapeDtypeStruct((1, 1), jnp.float32),
  )(acc_tc, acc_sc, g1, g2, cand)


def kernel(logits, candidates):
  b, c = logits.shape
  k = candidates.shape[1]
  cand = candidates.astype(jnp.int32)
  acc_tc, tab_tc = _stream_pass_tc(logits)
  acc_sc, tab_sc = _stream_pass_sc(logits)
  cp_tc = tab_tc.shape[1]
  cp_sc = tab_sc.shape[1]
  rows = jnp.arange(b, dtype=jnp.int32)[:, None]
  head_rows = C0 // (2 * CHUNK)
  p1 = jnp.where(cand >= C1, head_rows + (cand - C1) // (2 * CHUNK),
                 jnp.clip(cand // (2 * CHUNK), 0, head_rows - 1))
  idx1 = (rows * cp_tc + p1).reshape(b * k)
  idx2 = (rows * cp_sc
          + jnp.clip((cand - C0) // (2 * CHUNK), 0, cp_sc - 1)
          ).reshape(b * k)
  g1 = _gather_chunks(tab_tc.reshape(b * cp_tc, CHUNK), idx1)
  g2 = _gather_chunks(tab_sc.reshape(b * cp_sc, CHUNK), idx2)
  loss = _combine(acc_tc, acc_sc, g1.reshape(b, k * CHUNK),
                  g2.reshape(b, k * CHUNK), cand, c)
  return loss[0, 0]


# speedup vs baseline: 4.2774x; 4.2774x over previous
import functools

import jax
import jax.numpy as jnp
from jax import lax
from jax.experimental import pallas as pl
from jax.experimental.pallas import tpu as pltpu
from jax.experimental.pallas import tpu_sc as plsc

CHUNK = 128
LOG2E = 1.4426950408889634
LN2 = 0.6931471805599453


def _softplus_fast(x):
  e = jnp.exp2(jnp.abs(x) * (-LOG2E))
  return jnp.maximum(x, 0.0) + jnp.log2(1.0 + e) * LN2


def _stream_pass(logits, c_t=2048):
  b, c = logits.shape
  ncb = (c + c_t - 1) // c_t
  pairs = c_t // (2 * CHUNK)
  cp = ncb * pairs
  valid_last = c - (ncb - 1) * c_t

  def body(x_ref, acc_ref, tab_ref):
    j = pl.program_id(0)

    @pl.when(j == 0)
    def _():
      acc_ref[...] = jnp.zeros_like(acc_ref)

    x = x_ref[...]
    u = lax.bitcast_convert_type(x, jnp.uint32)
    for p in range(pairs):
      lo = u[:, 2 * p * CHUNK:(2 * p + 1) * CHUNK] >> 16
      hi = u[:, (2 * p + 1) * CHUNK:(2 * p + 2) * CHUNK] & jnp.uint32(
          0xFFFF0000)
      tab_ref[:, p, :] = lax.bitcast_convert_type(hi | lo, jnp.int32)
    y = _softplus_fast(x)

    def accum(y):
      acc = acc_ref[...]
      for s in range(c_t // 128):
        acc = acc + y[:, s * 128:(s + 1) * 128]
      acc_ref[...] = acc

    @pl.when(j < ncb - 1)
    def _():
      accum(y)

    @pl.when(j == ncb - 1)
    def _():
      col = lax.broadcasted_iota(jnp.int32, (b, c_t), 1)
      accum(jnp.where(col < valid_last, y, 0.0))

  return pl.pallas_call(
      body,
      grid=(ncb,),
      in_specs=[pl.BlockSpec((b, c_t), lambda j: (0, j))],
      out_specs=[
          pl.BlockSpec((b, 128), lambda j: (0, 0)),
          pl.BlockSpec((b, pairs, CHUNK), lambda j: (0, j, 0)),
      ],
      out_shape=[
          jax.ShapeDtypeStruct((b, 128), jnp.float32),
          jax.ShapeDtypeStruct((b, cp, CHUNK), jnp.int32),
      ],
  )(logits)


def _gather_chunks(table, chunk_idx):
  n = chunk_idx.shape[0]
  info = plsc.get_sparse_core_info()
  nw = info.num_cores * info.num_subcores
  per_w = n // nw
  assert n % (8 * nw) == 0
  mesh = plsc.VectorSubcoreMesh(core_axis_name="c", subcore_axis_name="s")

  @functools.partial(
      pl.kernel,
      mesh=mesh,
      out_type=jax.ShapeDtypeStruct((n, CHUNK), jnp.int32),
      scratch_types=[
          pltpu.VMEM((per_w,), jnp.int32),
          pltpu.VMEM((per_w, CHUNK), jnp.int32),
          pltpu.SemaphoreType.DMA,
      ],
  )
  def sc_kernel(tab_hbm, idx_hbm, out_hbm, idx_v, rows_v, sem):
    wid = lax.axis_index("s") * info.num_cores + lax.axis_index("c")
    base = wid * per_w
    pltpu.sync_copy(idx_hbm.at[pl.ds(base, per_w)], idx_v)
    pltpu.async_copy(tab_hbm.at[idx_v], rows_v, sem).wait()
    pltpu.sync_copy(rows_v, out_hbm.at[pl.ds(base, per_w)])

  return sc_kernel(table, chunk_idx)


def _combine(row_acc, g_rows, cand, c):
  b = row_acc.shape[0]
  k = cand.shape[1]

  def body(acc_ref, g_ref, cand_ref, out_ref):
    row_sum = jnp.sum(acc_ref[...], axis=1, keepdims=True)
    cd = cand_ref[...]
    gi = g_ref[...]
    lanes = lax.broadcasted_iota(jnp.int32, (b, CHUNK), 1)
    hi_mask = jnp.int32(-65536)
    gs = []
    for kk in range(k):
      cdk = cd[:, kk:kk + 1]
      sel = lanes == (cdk % CHUNK)
      row = jnp.where(sel, gi[:, kk * CHUNK:(kk + 1) * CHUNK], 0)
      packed = jnp.sum(row, axis=1, keepdims=True)
      half_hi = (cdk // CHUNK) % 2 == 1
      bits = jnp.where(half_hi, packed & hi_mask, packed << 16)
      gs.append(lax.bitcast_convert_type(bits, jnp.float32))
    pos = gs[0]
    for kk in range(1, k):
      pos = pos + gs[kk]
    pos = pos / k
    sub = _softplus_fast(gs[0])
    n_uniq = jnp.ones((b, 1), jnp.float32)
    for kk in range(1, k):
      w = jnp.ones((b, 1), jnp.float32)
      for jj in range(kk):
        w = w * (cd[:, kk:kk + 1] != cd[:, jj:jj + 1]).astype(jnp.float32)
      sub = sub + w * _softplus_fast(gs[kk])
      n_uniq = n_uniq + w
    neg = (row_sum - sub) / (c - n_uniq)
    per = _softplus_fast(-pos) + neg
    out_ref[0, 0] = jnp.sum(per) / b

  return pl.pallas_call(
      body,
      out_specs=pl.BlockSpec(memory_space=pltpu.SMEM),
      out_shape=jax.ShapeDtypeStruct((1, 1), jnp.float32),
  )(row_acc, g_rows, cand)


def kernel(logits, candidates):
  b, c = logits.shape
  k = candidates.shape[1]
  cand = candidates.astype(jnp.int32)
  row_acc, table = _stream_pass(logits)
  cp = table.shape[1]
  chunk_idx = (
      jnp.arange(b, dtype=jnp.int32)[:, None] * cp + cand // (2 * CHUNK)
  ).reshape(b * k)
  g = _gather_chunks(table.reshape(b * cp, CHUNK), chunk_idx)
  loss = _combine(row_acc, g.reshape(b, k * CHUNK), cand, c)
  return loss[0, 0]
